# baseline (speedup 1.0000x reference)
import functools
import jax
import jax.numpy as jnp
from jax import lax
from jax.experimental import pallas as pl
from jax.experimental.pallas import tpu as pltpu
from jax.experimental.pallas import tpu_sc as plsc

NUM_HEADS = 16
MAX_DISTANCE = 60
SEQ = 2048
A = 2040
LPAD = 4096


@functools.lru_cache(maxsize=1)
def _make_sc_kernel():
    mesh = plsc.VectorSubcoreMesh(
        core_axis_name="c", subcore_axis_name="s", num_cores=2, num_subcores=16
    )

    @functools.partial(
        pl.kernel,
        out_type=jax.ShapeDtypeStruct((NUM_HEADS * SEQ * SEQ,), jnp.float32),
        mesh=mesh,
        scratch_types=[
            pltpu.VMEM((8 * LPAD,), jnp.float32),
            pltpu.SemaphoreType.DMA,
        ],
    )
    def sc_kernel(w8_hbm, out_hbm, bank_v, sem):
        wid = lax.axis_index("s") * 2 + lax.axis_index("c")
        head = wid // 2
        half = wid % 2
        pltpu.sync_copy(w8_hbm.at[pl.ds(head * 8 * LPAD, 8 * LPAD)], bank_v)
        p0 = half * (SEQ // 2) // 8

        ngroups = (SEQ // 2) // 8

        def copies(p):
            off = A - 8 * p
            dst0 = (head * SEQ + 8 * p) * SEQ
            return [
                pltpu.make_async_copy(
                    bank_v.at[pl.ds(s * LPAD + off, SEQ)],
                    out_hbm.at[pl.ds(dst0 + s * SEQ, SEQ)],
                    sem,
                )
                for s in range(8)
            ]

        def group(g, carry):
            for c in copies(p0 + g):
                c.start()

            @pl.when(g > 0)
            def _():
                for c in copies(p0 + g - 1):
                    c.wait()

            return carry

        lax.fori_loop(0, ngroups, group, 0)
        for c in copies(p0 + ngroups - 1):
            c.wait()

    return sc_kernel


@jax.jit
def kernel(seq_len, table):
    del seq_len
    m = jnp.arange(LPAD)
    s = jnp.arange(8)
    d = m[None, :] - s[:, None] - A
    idx = jnp.clip(d, -MAX_DISTANCE, MAX_DISTANCE) + MAX_DISTANCE
    w8 = jnp.transpose(table[idx], (2, 0, 1))
    out = _make_sc_kernel()(w8.reshape(-1))
    return out.reshape(NUM_HEADS, SEQ, SEQ)


# speedup vs baseline: 1.0003x; 1.0003x over previous
import functools
import jax
import jax.numpy as jnp
from jax import lax
from jax.experimental import pallas as pl
from jax.experimental.pallas import tpu as pltpu
from jax.experimental.pallas import tpu_sc as plsc

NUM_HEADS = 16
MAX_DISTANCE = 60
SEQ = 2048
A = 2040
LPAD = 4096


@functools.lru_cache(maxsize=1)
def _make_sc_kernel():
    mesh = plsc.VectorSubcoreMesh(
        core_axis_name="c", subcore_axis_name="s", num_cores=2, num_subcores=16
    )

    @functools.partial(
        pl.kernel,
        out_type=jax.ShapeDtypeStruct((NUM_HEADS * SEQ // 8, 8, SEQ), jnp.float32),
        mesh=mesh,
        scratch_types=[
            pltpu.VMEM((8, LPAD), jnp.float32),
            pltpu.SemaphoreType.DMA,
        ],
        compiler_params=pltpu.CompilerParams(use_tc_tiling_on_sc=False),
    )
    def sc_kernel(w8_hbm, out_hbm, bank_v, sem):
        wid = lax.axis_index("s") * 2 + lax.axis_index("c")
        head = wid // 2
        half = wid % 2
        pltpu.sync_copy(w8_hbm.at[head], bank_v)
        p0 = half * (SEQ // 2) // 8
        ngroups = (SEQ // 2) // 8

        def copy(p):
            return pltpu.make_async_copy(
                bank_v.at[:, pl.ds(A - 8 * p, SEQ)],
                out_hbm.at[head * (SEQ // 8) + p],
                sem,
            )

        def group(g, carry):
            copy(p0 + g).start()

            @pl.when(g > 0)
            def _():
                copy(p0 + g - 1).wait()

            return carry

        lax.fori_loop(0, ngroups, group, 0)
        copy(p0 + ngroups - 1).wait()

    return sc_kernel


@jax.jit
def kernel(seq_len, table):
    del seq_len
    m = jnp.arange(LPAD)
    s = jnp.arange(8)
    d = m[None, :] - s[:, None] - A
    idx = jnp.clip(d, -MAX_DISTANCE, MAX_DISTANCE) + MAX_DISTANCE
    w8 = jnp.transpose(table[idx], (2, 0, 1))
    out = _make_sc_kernel()(w8)
    return out.reshape(NUM_HEADS, SEQ, SEQ)


# speedup vs baseline: 2.4030x; 2.4022x over previous
import jax
import jax.numpy as jnp
from jax.experimental import pallas as pl

NUM_HEADS = 16
MAX_DISTANCE = 60
SEQ = 2048
ROWS_PER_BLOCK = 2048
A = 2040
LPAD = 4096


def _toeplitz_body(w8_ref, out_ref):
    R = ROWS_PER_BLOCK
    q = pl.program_id(1)
    base = pl.multiple_of(SEQ - R * (q + 1), 128)
    chunk = w8_ref[0, :, pl.ds(base, R + SEQ)]
    for t in range(R // 8):
        lo = R - 8 - 8 * t
        out_ref[0, 8 * t:8 * t + 8, :] = chunk[:, lo:lo + SEQ]


@jax.jit
def kernel(seq_len, table):
    del seq_len
    m = jnp.arange(LPAD)
    s = jnp.arange(8)
    d = m[None, :] - s[:, None] - A
    idx = jnp.clip(d, -MAX_DISTANCE, MAX_DISTANCE) + MAX_DISTANCE
    w8 = jnp.transpose(table[idx], (2, 0, 1))

    return pl.pallas_call(
        _toeplitz_body,
        grid=(NUM_HEADS, SEQ // ROWS_PER_BLOCK),
        in_specs=[pl.BlockSpec((1, 8, LPAD), lambda h, q: (h, 0, 0))],
        out_specs=pl.BlockSpec((1, ROWS_PER_BLOCK, SEQ), lambda h, q: (h, q, 0)),
        out_shape=jax.ShapeDtypeStruct((NUM_HEADS, SEQ, SEQ), jnp.float32),
    )(w8)


# speedup vs baseline: 2.4244x; 1.0089x over previous
import jax
import jax.numpy as jnp
from jax.experimental import pallas as pl

NUM_HEADS = 16
MAX_DISTANCE = 60
SEQ = 2048
ROWS_PER_BLOCK = 1024
A = 2040
LPAD = 4096


def _toeplitz_body(w8_ref, out_ref):
    R = ROWS_PER_BLOCK
    q = pl.program_id(1)
    base = pl.multiple_of(SEQ - R * (q + 1), 128)
    chunk = w8_ref[0, :, pl.ds(base, R + SEQ)]
    for t in range(R // 8):
        lo = R - 8 - 8 * t
        out_ref[0, 8 * t:8 * t + 8, :] = chunk[:, lo:lo + SEQ]


@jax.jit
def kernel(seq_len, table):
    del seq_len
    m = jnp.arange(LPAD)
    s = jnp.arange(8)
    d = m[None, :] - s[:, None] - A
    idx = jnp.clip(d, -MAX_DISTANCE, MAX_DISTANCE) + MAX_DISTANCE
    w8 = jnp.transpose(table[idx], (2, 0, 1))

    return pl.pallas_call(
        _toeplitz_body,
        grid=(NUM_HEADS, SEQ // ROWS_PER_BLOCK),
        in_specs=[pl.BlockSpec((1, 8, LPAD), lambda h, q: (h, 0, 0))],
        out_specs=pl.BlockSpec((1, ROWS_PER_BLOCK, SEQ), lambda h, q: (h, q, 0)),
        out_shape=jax.ShapeDtypeStruct((NUM_HEADS, SEQ, SEQ), jnp.float32),
    )(w8)
